# Initial kernel scaffold; baseline (speedup 1.0000x reference)
import jax
import jax.numpy as jnp
from jax.experimental import pallas as pl


def kernel(queries, keys):
    raise NotImplementedError("write your pallas kernel here")


# speedup vs baseline: 14.5535x; 14.5535x over previous
import functools

import jax
import jax.numpy as jnp
from jax.experimental import pallas as pl

_Q_BLK = 784
_K_PAD = 10240
_CK = 2048
_N_CHUNK = _K_PAD // _CK
_PAD_VAL = 100.0


def _knn_body(q_ref, kt_ref, map_ref, max_ref):
    q = q_ref[...]
    qb = q.astype(jnp.bfloat16)

    def chunk(c, mins):
        kt = kt_ref[:, pl.ds(c * _CK, _CK)]
        ksq = jnp.sum(kt * kt, axis=0, keepdims=True)
        prod = jax.lax.dot_general(
            qb, kt.astype(jnp.bfloat16),
            (((1,), (0,)), ((), ())),
            preferred_element_type=jnp.float32)
        t = ksq - 2.0 * prod
        return jnp.minimum(mins, jnp.min(t, axis=1, keepdims=True))

    mins = jax.lax.fori_loop(
        0, _N_CHUNK, chunk,
        jnp.full((_Q_BLK, 1), jnp.inf, jnp.float32))
    qsq = jnp.sum(q * q, axis=1, keepdims=True)
    dist = jnp.sqrt(jnp.maximum(mins + qsq, 1e-12))
    map_ref[0, :, :] = dist
    max_ref[0, :, :] = jnp.max(dist, axis=(0, 1), keepdims=True)


@functools.partial(jax.jit, static_argnames=())
def kernel(queries, keys):
    n_img = queries.shape[0] // _Q_BLK
    kt = jnp.pad(keys.T, ((0, 0), (0, _K_PAD - keys.shape[0])),
                 constant_values=_PAD_VAL)
    amap, amax = pl.pallas_call(
        _knn_body,
        grid=(n_img,),
        in_specs=[
            pl.BlockSpec((_Q_BLK, queries.shape[1]), lambda i: (i, 0)),
            pl.BlockSpec(kt.shape, lambda i: (0, 0)),
        ],
        out_specs=[
            pl.BlockSpec((1, _Q_BLK, 1), lambda i: (i, 0, 0)),
            pl.BlockSpec((1, 1, 1), lambda i: (i, 0, 0)),
        ],
        out_shape=[
            jax.ShapeDtypeStruct((n_img, _Q_BLK, 1), jnp.float32),
            jax.ShapeDtypeStruct((n_img, 1, 1), jnp.float32),
        ],
    )(queries, kt)
    return amax.reshape(n_img), amap.reshape(n_img, 28, 28)


# speedup vs baseline: 14.7803x; 1.0156x over previous
import functools

import jax
import jax.numpy as jnp
from jax.experimental import pallas as pl
from jax.experimental.pallas import tpu as pltpu

_Q_BLK = 784
_K_PAD = 10240
_CK = 2048
_N_CHUNK = _K_PAD // _CK
_PAD_VAL = 100.0


def _knn_body(q_ref, kt_ref, map_ref, max_ref):
    q = q_ref[...]
    qb = (-2.0 * q).astype(jnp.bfloat16)

    def chunk(c, acc):
        kt = kt_ref[:, pl.ds(c * _CK, _CK)]
        ksq = jnp.sum(kt * kt, axis=0, keepdims=True)
        prod = jax.lax.dot_general(
            qb, kt.astype(jnp.bfloat16),
            (((1,), (0,)), ((), ())),
            preferred_element_type=jnp.float32)
        t = prod + ksq
        for s in range(_CK // 128):
            acc = jnp.minimum(acc, t[:, s * 128:(s + 1) * 128])
        return acc

    acc = jax.lax.fori_loop(
        0, _N_CHUNK, chunk,
        jnp.full((_Q_BLK, 128), jnp.inf, jnp.float32))
    mins = jnp.min(acc, axis=1, keepdims=True)
    qsq = jnp.sum(q * q, axis=1, keepdims=True)
    dist = jnp.sqrt(jnp.maximum(mins + qsq, 1e-12))
    map_ref[0, :, :] = dist
    max_ref[0, :, :] = jnp.max(dist, axis=(0, 1), keepdims=True)


@functools.partial(jax.jit, static_argnames=())
def kernel(queries, keys):
    n_img = queries.shape[0] // _Q_BLK
    kt = jnp.pad(keys.T, ((0, 0), (0, _K_PAD - keys.shape[0])),
                 constant_values=_PAD_VAL)
    amap, amax = pl.pallas_call(
        _knn_body,
        grid=(n_img,),
        in_specs=[
            pl.BlockSpec((_Q_BLK, queries.shape[1]), lambda i: (i, 0)),
            pl.BlockSpec(kt.shape, lambda i: (0, 0)),
        ],
        out_specs=[
            pl.BlockSpec((1, _Q_BLK, 1), lambda i: (i, 0, 0)),
            pl.BlockSpec((1, 1, 1), lambda i: (i, 0, 0)),
        ],
        out_shape=[
            jax.ShapeDtypeStruct((n_img, _Q_BLK, 1), jnp.float32),
            jax.ShapeDtypeStruct((n_img, 1, 1), jnp.float32),
        ],
        compiler_params=pltpu.CompilerParams(
            dimension_semantics=("parallel",)),
    )(queries, kt)
    return amax.reshape(n_img), amap.reshape(n_img, 28, 28)
